# baseline (speedup 1.0000x reference)
import functools

import jax
import jax.numpy as jnp
from jax import lax
from jax.experimental import pallas as pl
from jax.experimental.pallas import tpu as pltpu
from jax.experimental.pallas import tpu_sc as plsc

N = 4 * 4096 * 2048
NW = 32
CH = 16384


def _take16(v, idx):
    return lax.gather(
        v,
        idx[:, None],
        dimension_numbers=lax.GatherDimensionNumbers(
            offset_dims=(), collapsed_slice_dims=(0,), start_index_map=(0,)),
        slice_sizes=(1,),
        mode=lax.GatherScatterMode.PROMISE_IN_BOUNDS,
    )


def _build(n=N, ch=CH, interpret=False):
    per_w = n // NW
    steps = ch // 32

    nchunks = per_w // ch
    assert nchunks % 2 == 0

    @functools.partial(
        pl.kernel,
        out_type=jax.ShapeDtypeStruct((n,), jnp.float32),
        mesh=plsc.VectorSubcoreMesh(core_axis_name="c", subcore_axis_name="s"),
        scratch_types=[
            pltpu.VMEM((ch,), jnp.float32),
            pltpu.VMEM((ch,), jnp.float32),
            pltpu.VMEM((ch,), jnp.float32),
            pltpu.VMEM((ch,), jnp.float32),
            pltpu.SemaphoreType.DMA,
            pltpu.SemaphoreType.DMA,
            pltpu.SemaphoreType.DMA,
            pltpu.SemaphoreType.DMA,
        ],
        compiler_params=pltpu.CompilerParams(needs_layout_passes=False),
        interpret=interpret,
    )
    def sc_mask48(x_hbm, out_hbm, xin0, xin1, xout0, xout1,
                  si0, si1, so0, so1):
        cid = lax.axis_index("c")
        sid = lax.axis_index("s")
        wid = sid * 2 + cid
        base = wid * per_w
        xin = (xin0, xin1)
        xout = (xout0, xout1)
        sem_in = (si0, si1)
        sem_out = (so0, so1)

        lane = lax.iota(jnp.int32, 16)
        q = lane & 3
        rots = [(lane - q) + ((q + k) & 3) for k in (1, 2, 3)]
        ties = [((q + k) & 3) < q for k in (1, 2, 3)]
        idx_e = lane * 2
        idx_o = idx_e + 1
        zero = jnp.zeros((16,), jnp.float32)

        def make_step(src, dst):
            def step(j, carry):
                b32 = j * 32
                ie = b32 + idx_e
                io = b32 + idx_o
                a = plsc.load_gather(src, [ie])
                b = plsc.load_gather(src, [io])
                sq = a * a + b * b
                nb1 = _take16(sq, rots[0])
                nb2 = _take16(sq, rots[1])
                nb3 = _take16(sq, rots[2])
                c1 = (nb1 < sq) | ((nb1 == sq) & ties[0])
                c2 = (nb2 < sq) | ((nb2 == sq) & ties[1])
                c3 = (nb3 < sq) | ((nb3 == sq) & ties[2])
                keep = (c1 & c2) | (c1 & c3) | (c2 & c3)
                plsc.store_scatter(dst, [ie], jnp.where(keep, a, zero))
                plsc.store_scatter(dst, [io], jnp.where(keep, b, zero))
                return carry
            return step

        for b in (0, 1):
            pltpu.async_copy(x_hbm.at[pl.ds(base + b * ch, ch)],
                             xin[b], sem_in[b])

        def chunk_pair(cp, carry):
            for b in (0, 1):
                ci = cp * 2 + b
                off = base + ci * ch
                pltpu.make_async_copy(x_hbm.at[pl.ds(off, ch)],
                                      xin[b], sem_in[b]).wait()
                @pl.when(ci >= 2)
                def _():
                    pltpu.make_async_copy(
                        xout[b], out_hbm.at[pl.ds(off, ch)], sem_out[b]).wait()
                step_fn = make_step(xin[b], xout[b])
                plsc.parallel_loop(0, steps, 1, unroll=8)(
                    lambda j, fn=step_fn: fn(j, None))
                pltpu.async_copy(xout[b], out_hbm.at[pl.ds(off, ch)],
                                 sem_out[b])
                @pl.when(ci + 2 < nchunks)
                def _():
                    pltpu.async_copy(x_hbm.at[pl.ds(off + 2 * ch, ch)],
                                     xin[b], sem_in[b])
            return carry

        lax.fori_loop(0, nchunks // 2, chunk_pair, 0)

        for b in (0, 1):
            off = base + (nchunks - 2 + b) * ch
            pltpu.make_async_copy(xout[b], out_hbm.at[pl.ds(off, ch)],
                                  sem_out[b]).wait()

    return sc_mask48


_sc_mask48 = _build()


def kernel(x):
    return _sc_mask48(x.reshape(-1)).reshape(x.shape)


# speedup vs baseline: 6.5502x; 6.5502x over previous
import functools

import jax
import jax.numpy as jnp
from jax import lax
from jax.experimental import pallas as pl
from jax.experimental.pallas import tpu as pltpu
from jax.experimental.pallas import tpu_sc as plsc

B, R, C = 4, 4096, 2048
NW = 32
ROWS_W = (B * R) // NW
BAND = 8


def _take16(v, idx):
    return lax.gather(
        v,
        idx[:, None],
        dimension_numbers=lax.GatherDimensionNumbers(
            offset_dims=(), collapsed_slice_dims=(0,), start_index_map=(0,)),
        slice_sizes=(1,),
        mode=lax.GatherScatterMode.PROMISE_IN_BOUNDS,
    )


def _build(interpret=False):
    nchunks = ROWS_W // BAND
    steps = BAND * C // 32

    @functools.partial(
        pl.kernel,
        out_type=jax.ShapeDtypeStruct((B, R, C), jnp.float32),
        mesh=plsc.VectorSubcoreMesh(core_axis_name="c", subcore_axis_name="s"),
        scratch_types=[
            pltpu.VMEM((BAND, C), jnp.float32),
            pltpu.VMEM((BAND, C), jnp.float32),
            pltpu.VMEM((BAND, C), jnp.float32),
            pltpu.VMEM((BAND, C), jnp.float32),
            pltpu.SemaphoreType.DMA,
            pltpu.SemaphoreType.DMA,
            pltpu.SemaphoreType.DMA,
            pltpu.SemaphoreType.DMA,
        ],
        compiler_params=pltpu.CompilerParams(
            needs_layout_passes=False, use_tc_tiling_on_sc=True),
        interpret=interpret,
    )
    def sc_mask48(x_hbm, out_hbm, xin0, xin1, xout0, xout1,
                  si0, si1, so0, so1):
        cid = lax.axis_index("c")
        sid = lax.axis_index("s")
        wid = sid * 2 + cid
        bi = wid // 8
        row0 = (wid % 8) * ROWS_W
        xin = (xin0, xin1)
        xout = (xout0, xout1)
        sem_in = (si0, si1)
        sem_out = (so0, so1)

        lane = lax.iota(jnp.int32, 16)
        q = lane & 3
        rots = [(lane - q) + ((q + k) & 3) for k in (1, 2, 3)]
        ties = [((q + k) & 3) < q for k in (1, 2, 3)]
        idx_e = lane * 2
        idx_o = idx_e + 1
        zero = jnp.zeros((16,), jnp.float32)
        zeroi = jnp.zeros((16,), jnp.int32)

        def make_step(src, dst):
            def step(j, carry):
                r = j // (C // 32)
                cb = (j % (C // 32)) * 32
                rvec = zeroi + r
                ie = cb + idx_e
                io = cb + idx_o
                a = plsc.load_gather(src, [rvec, ie])
                b = plsc.load_gather(src, [rvec, io])
                sq = a * a + b * b
                nb1 = _take16(sq, rots[0])
                nb2 = _take16(sq, rots[1])
                nb3 = _take16(sq, rots[2])
                c1 = (nb1 < sq) | ((nb1 == sq) & ties[0])
                c2 = (nb2 < sq) | ((nb2 == sq) & ties[1])
                c3 = (nb3 < sq) | ((nb3 == sq) & ties[2])
                keep = (c1 & c2) | (c1 & c3) | (c2 & c3)
                plsc.store_scatter(dst, [rvec, ie], jnp.where(keep, a, zero))
                plsc.store_scatter(dst, [rvec, io], jnp.where(keep, b, zero))
                return carry
            return step

        def in_slice(ci):
            return x_hbm.at[bi, pl.ds(row0 + ci * BAND, BAND)]

        def out_slice(ci):
            return out_hbm.at[bi, pl.ds(row0 + ci * BAND, BAND)]

        for b in (0, 1):
            pltpu.async_copy(in_slice(b), xin[b], sem_in[b])

        def chunk_pair(cp, carry):
            for b in (0, 1):
                ci = cp * 2 + b
                pltpu.make_async_copy(in_slice(ci), xin[b], sem_in[b]).wait()
                @pl.when(ci >= 2)
                def _():
                    pltpu.make_async_copy(xout[b], out_slice(ci),
                                          sem_out[b]).wait()
                step_fn = make_step(xin[b], xout[b])
                plsc.parallel_loop(0, steps, 1, unroll=4)(
                    lambda j, fn=step_fn: fn(j, None))
                pltpu.async_copy(xout[b], out_slice(ci), sem_out[b])
                @pl.when(ci + 2 < nchunks)
                def _():
                    pltpu.async_copy(in_slice(ci + 2), xin[b], sem_in[b])
            return carry

        lax.fori_loop(0, nchunks // 2, chunk_pair, 0)

        for b in (0, 1):
            pltpu.make_async_copy(xout[b], out_slice(nchunks - 2 + b),
                                  sem_out[b]).wait()

    return sc_mask48


_sc_mask48 = _build()


def kernel(x):
    return _sc_mask48(x)


# speedup vs baseline: 9.5336x; 1.4555x over previous
import functools

import jax
import jax.numpy as jnp
from jax import lax
from jax.experimental import pallas as pl
from jax.experimental.pallas import tpu as pltpu
from jax.experimental.pallas import tpu_sc as plsc

B, R, C = 4, 4096, 2048
NW = 32
ROWS_W = (B * R) // NW
BAND = 8


def _take16(v, idx):
    return lax.gather(
        v,
        idx[:, None],
        dimension_numbers=lax.GatherDimensionNumbers(
            offset_dims=(), collapsed_slice_dims=(0,), start_index_map=(0,)),
        slice_sizes=(1,),
        mode=lax.GatherScatterMode.PROMISE_IN_BOUNDS,
    )


def _build(interpret=False):
    nchunks = ROWS_W // BAND
    steps = BAND * C // 32

    @functools.partial(
        pl.kernel,
        out_type=jax.ShapeDtypeStruct((B, R, C), jnp.float32),
        mesh=plsc.VectorSubcoreMesh(core_axis_name="c", subcore_axis_name="s"),
        scratch_types=[
            pltpu.VMEM((BAND, C), jnp.float32),
            pltpu.VMEM((BAND, C), jnp.float32),
            pltpu.VMEM((BAND, C), jnp.float32),
            pltpu.VMEM((BAND, C), jnp.float32),
            pltpu.SemaphoreType.DMA,
            pltpu.SemaphoreType.DMA,
            pltpu.SemaphoreType.DMA,
            pltpu.SemaphoreType.DMA,
        ],
        compiler_params=pltpu.CompilerParams(
            needs_layout_passes=False, use_tc_tiling_on_sc=True),
        interpret=interpret,
    )
    def sc_mask48(x_hbm, out_hbm, xin0, xin1, xout0, xout1,
                  si0, si1, so0, so1):
        cid = lax.axis_index("c")
        sid = lax.axis_index("s")
        wid = sid * 2 + cid
        bi = wid // 8
        row0 = (wid % 8) * ROWS_W
        xin = (xin0, xin1)
        xout = (xout0, xout1)
        sem_in = (si0, si1)
        sem_out = (so0, so1)

        lane = lax.iota(jnp.int32, 16)
        q = lane & 3
        rots = [(lane - q) + ((q + k) & 3) for k in (1, 2, 3)]
        idx_e = lane * 2
        idx_o = idx_e + 1
        zero = jnp.zeros((16,), jnp.float32)
        zeroi = jnp.zeros((16,), jnp.int32)

        def make_step(src, dst):
            def step(j, carry):
                r = j // (C // 32)
                cb = (j % (C // 32)) * 32
                rvec = zeroi + r
                ie = cb + idx_e
                io = cb + idx_o
                a = plsc.load_gather(src, [rvec, ie])
                b = plsc.load_gather(src, [rvec, io])
                sq = a * a + b * b
                nb1 = _take16(sq, rots[0])
                nb2 = _take16(sq, rots[1])
                nb3 = _take16(sq, rots[2])
                c1 = nb1 < sq
                c2 = nb2 < sq
                c3 = nb3 < sq
                keep = (c1 & c2) | (c1 & c3) | (c2 & c3)
                plsc.store_scatter(dst, [rvec, ie], jnp.where(keep, a, zero))
                plsc.store_scatter(dst, [rvec, io], jnp.where(keep, b, zero))
                return carry
            return step

        def in_slice(ci):
            return x_hbm.at[bi, pl.ds(row0 + ci * BAND, BAND)]

        def out_slice(ci):
            return out_hbm.at[bi, pl.ds(row0 + ci * BAND, BAND)]

        for b in (0, 1):
            pltpu.async_copy(in_slice(b), xin[b], sem_in[b])

        def chunk_pair(cp, carry):
            for b in (0, 1):
                ci = cp * 2 + b
                pltpu.make_async_copy(in_slice(ci), xin[b], sem_in[b]).wait()
                @pl.when(ci >= 2)
                def _():
                    pltpu.make_async_copy(xout[b], out_slice(ci),
                                          sem_out[b]).wait()
                step_fn = make_step(xin[b], xout[b])
                plsc.parallel_loop(0, steps, 1, unroll=4)(
                    lambda j, fn=step_fn: fn(j, None))
                pltpu.async_copy(xout[b], out_slice(ci), sem_out[b])
                @pl.when(ci + 2 < nchunks)
                def _():
                    pltpu.async_copy(in_slice(ci + 2), xin[b], sem_in[b])
            return carry

        lax.fori_loop(0, nchunks // 2, chunk_pair, 0)

        for b in (0, 1):
            pltpu.make_async_copy(xout[b], out_slice(nchunks - 2 + b),
                                  sem_out[b]).wait()

    return sc_mask48


_sc_mask48 = _build()


def kernel(x):
    return _sc_mask48(x)
